# baseline (speedup 1.0000x reference)
import functools
import numpy as np
import jax
import jax.numpy as jnp
from jax import lax
from jax.experimental import pallas as pl
from jax.experimental.pallas import tpu as pltpu
from jax.experimental.pallas import tpu_sc as plsc

B = 16384
D_IN = 69
D_OUT = 27
NC = 2
NS = 16
NW = NC * NS
RW = B // NW
SUB = 128
L = 16
NCHUNK = SUB // L

_TP = float(np.float32(0.6366197723675814))
_BIG = float(np.float32(12582912.0))
_C1 = float(np.float32(np.round(np.pi / 2 * 2048.0) / 2048.0))
_C2 = float(np.float32(np.pi / 2 - np.float64(np.float32(np.round(np.pi / 2 * 2048.0) / 2048.0))))
_S1 = float(np.float32(-1.6666654611e-1))
_S2 = float(np.float32(8.3321608736e-3))
_S3 = float(np.float32(-1.9515295891e-4))
_CC1 = float(np.float32(4.166664568298827e-2))
_CC2 = float(np.float32(-1.388731625493765e-3))
_CC3 = float(np.float32(2.443315711809948e-5))


def _sincos(x):
    t = x * _TP
    n = (t + _BIG) - _BIG
    r = (x - n * _C1) - n * _C2
    ni = n.astype(jnp.int32)
    z = r * r
    s = r + r * z * (_S1 + z * (_S2 + z * _S3))
    c = 1.0 + z * (-0.5 + z * (_CC1 + z * (_CC2 + z * _CC3)))
    k = lax.bitwise_and(ni, 3)
    swap = lax.bitwise_and(k, 1) == 1
    sv = jnp.where(swap, c, s)
    cv = jnp.where(swap, s, c)
    sv = jnp.where(lax.bitwise_and(k, 2) == 2, -sv, sv)
    cv = jnp.where(lax.bitwise_and(k + 1, 2) == 2, -cv, cv)
    return sv, cv


def _mixer_body(obs_hbm, out_hbm, obs_a, obs_b, res_a, res_b,
                sia, sib, soa, sob):
    cid = lax.axis_index("c")
    sid = lax.axis_index("s")
    wid = sid * NC + cid
    iota = lax.iota(jnp.int32, L)

    NBLK = RW // SUB
    obs_bufs = [obs_a, obs_b]
    res_bufs = [res_a, res_b]
    sin = [sia, sib]
    sout = [soa, sob]

    def start_in(blk):
        base = wid * RW + blk * SUB
        return pltpu.async_copy(
            obs_hbm.at[pl.ds(base, SUB), :],
            obs_bufs[blk % 2].at[:, 0:D_IN], sin[blk % 2])

    def start_out(blk):
        base = wid * RW + blk * SUB
        return pltpu.async_copy(
            res_bufs[blk % 2].at[:, 0:D_OUT],
            out_hbm.at[pl.ds(base, SUB), :], sout[blk % 2])

    in_h = {0: start_in(0), 1: start_in(1)}
    out_h = {}
    for blk in range(NBLK):
        obs_v = obs_bufs[blk % 2]
        out_v = res_bufs[blk % 2]
        in_h[blk].wait()
        if blk >= 2:
            out_h[blk - 2].wait()

        def col(rows, c, obs_v=obs_v):
            return plsc.load_gather(
                obs_v, [rows, jnp.full((L,), c, jnp.int32)])

        def putc(rows, cvec, x, out_v=out_v):
            plsc.store_scatter(out_v, [rows, cvec], x)

        def put(rows, c, x):
            putc(rows, jnp.full((L,), c, jnp.int32), x)

        @plsc.parallel_loop(0, NCHUNK, unroll=4)
        def body(i):
            rows = i * L + iota
            c2 = []
            for k in range(7):
                x = col(rows, 13 + 7 * k)
                y = col(rows, 14 + 7 * k)
                z = col(rows, 15 + 7 * k)
                c2.append(x * x + y * y + z * z)
            zero = jnp.zeros((L,), jnp.int32)
            one = jnp.full((L,), 1, jnp.int32)
            rk = [zero] * 7
            for j in range(7):
                for k in range(j + 1, 7):
                    m = c2[j] <= c2[k]
                    rk[k] = rk[k] + jnp.where(m, one, zero)
                    rk[j] = rk[j] + jnp.where(m, zero, one)

            sa, ca = _sincos(col(rows, 3))
            sb, cb = _sincos(col(rows, 4))
            sc, cc = _sincos(col(rows, 5))
            r00 = cb * ca
            r01 = -cb * sa
            r02 = sb
            scsb = sc * sb
            ccsb = cc * sb
            r10 = cc * sa + scsb * ca
            r11 = cc * ca - scsb * sa
            r12 = -sc * cb
            r20 = sc * sa - ccsb * ca
            r21 = sc * ca + ccsb * sa
            r22 = cc * cb
            pos_x = col(rows, 0)
            pos_y = col(rows, 1)
            pos_z = col(rows, 2)

            def matvec(x, y, z):
                wx = pos_x - (r00 * x + r10 * y + r20 * z)
                wy = pos_y - (r01 * x + r11 * y + r21 * z)
                wz = pos_z - (r02 * x + r12 * y + r22 * z)
                return wx, wy, wz

            put(rows, 0, pos_x)
            put(rows, 1, pos_y)
            put(rows, 2, pos_z)
            for k in range(7):
                wx, wy, wz = matvec(col(rows, 13 + 7 * k),
                                    col(rows, 14 + 7 * k),
                                    col(rows, 15 + 7 * k))
                cbase = 3 + 3 * rk[k]
                putc(rows, cbase, wx)
                putc(rows, cbase + 1, wy)
                putc(rows, cbase + 2, wz)
            wx, wy, wz = matvec(col(rows, 62), col(rows, 63), col(rows, 64))
            put(rows, 24, wx)
            put(rows, 25, wy)
            put(rows, 26, wz)

        out_h[blk] = start_out(blk)
        if blk + 2 < NBLK:
            in_h[blk + 2] = start_in(blk + 2)

    out_h[NBLK - 2].wait()
    out_h[NBLK - 1].wait()


@jax.jit
def kernel(obs):
    mesh = plsc.VectorSubcoreMesh(core_axis_name="c", subcore_axis_name="s")
    f = functools.partial(
        pl.kernel,
        mesh=mesh,
        out_type=jax.ShapeDtypeStruct((B, D_OUT), jnp.float32),
        scratch_types=[
            pltpu.VMEM((SUB, D_IN), jnp.float32),
            pltpu.VMEM((SUB, D_IN), jnp.float32),
            pltpu.VMEM((SUB, D_OUT), jnp.float32),
            pltpu.VMEM((SUB, D_OUT), jnp.float32),
            pltpu.SemaphoreType.DMA,
            pltpu.SemaphoreType.DMA,
            pltpu.SemaphoreType.DMA,
            pltpu.SemaphoreType.DMA,
        ],
        compiler_params=pltpu.CompilerParams(needs_layout_passes=False, use_tc_tiling_on_sc=True),
    )(_mixer_body)
    return f(obs)


# speedup vs baseline: 1.1454x; 1.1454x over previous
import functools
import numpy as np
import jax
import jax.numpy as jnp
from jax import lax
from jax.experimental import pallas as pl
from jax.experimental.pallas import tpu as pltpu
from jax.experimental.pallas import tpu_sc as plsc

B = 16384
D_IN = 69
D_OUT = 27
NC = 2
NS = 16
NW = NC * NS
RW = B // NW
SUB = 128
L = 16
NCHUNK = SUB // L

_TP = float(np.float32(0.6366197723675814))
_BIG = float(np.float32(12582912.0))
_C1 = float(np.float32(np.round(np.pi / 2 * 2048.0) / 2048.0))
_C2 = float(np.float32(np.pi / 2 - np.float64(np.float32(np.round(np.pi / 2 * 2048.0) / 2048.0))))
_S1 = float(np.float32(-1.6666654611e-1))
_S2 = float(np.float32(8.3321608736e-3))
_S3 = float(np.float32(-1.9515295891e-4))
_CC1 = float(np.float32(4.166664568298827e-2))
_CC2 = float(np.float32(-1.388731625493765e-3))
_CC3 = float(np.float32(2.443315711809948e-5))


def _sincos(x):
    t = x * _TP
    n = (t + _BIG) - _BIG
    r = (x - n * _C1) - n * _C2
    ni = n.astype(jnp.int32)
    z = r * r
    s = r + r * z * (_S1 + z * (_S2 + z * _S3))
    c = 1.0 + z * (-0.5 + z * (_CC1 + z * (_CC2 + z * _CC3)))
    k = lax.bitwise_and(ni, 3)
    swap = lax.bitwise_and(k, 1) == 1
    sv = jnp.where(swap, c, s)
    cv = jnp.where(swap, s, c)
    sv = jnp.where(lax.bitwise_and(k, 2) == 2, -sv, sv)
    cv = jnp.where(lax.bitwise_and(k + 1, 2) == 2, -cv, cv)
    return sv, cv


def _mixer_body(obs_hbm, out_hbm, obs_a, obs_b, res_a, res_b,
                sia, sib, soa, sob):
    cid = lax.axis_index("c")
    sid = lax.axis_index("s")
    wid = sid * NC + cid
    iota = lax.iota(jnp.int32, L)

    NBLK = RW // SUB
    obs_bufs = [obs_a, obs_b]
    res_bufs = [res_a, res_b]
    sin = [sia, sib]
    sout = [soa, sob]

    def start_in(blk):
        base = wid * RW + blk * SUB
        return pltpu.async_copy(
            obs_hbm.at[pl.ds(base, SUB), :],
            obs_bufs[blk % 2].at[:, 0:D_IN], sin[blk % 2])

    def start_out(blk):
        base = wid * RW + blk * SUB
        return pltpu.async_copy(
            res_bufs[blk % 2].at[:, 0:D_OUT],
            out_hbm.at[pl.ds(base, SUB), :], sout[blk % 2])

    in_h = {0: start_in(0), 1: start_in(1)}
    out_h = {}
    for blk in range(NBLK):
        obs_v = obs_bufs[blk % 2]
        out_v = res_bufs[blk % 2]
        in_h[blk].wait()
        if blk >= 2:
            out_h[blk - 2].wait()

        def col(rows, c, obs_v=obs_v):
            return plsc.load_gather(
                obs_v, [rows, jnp.full((L,), c, jnp.int32)])

        def putc(rows, cvec, x, out_v=out_v):
            plsc.store_scatter(out_v, [rows, cvec], x)

        def put(rows, c, x):
            putc(rows, jnp.full((L,), c, jnp.int32), x)

        @plsc.parallel_loop(0, NCHUNK, unroll=2)
        def body(i):
            rows = i * L + iota
            px, py, pz, c2 = [], [], [], []
            for k in range(7):
                x = col(rows, 13 + 7 * k)
                y = col(rows, 14 + 7 * k)
                z = col(rows, 15 + 7 * k)
                px.append(x)
                py.append(y)
                pz.append(z)
                c2.append(x * x + y * y + z * z)
            zero = jnp.zeros((L,), jnp.int32)
            one = jnp.full((L,), 1, jnp.int32)
            rk = [zero] * 7
            for j in range(7):
                for k in range(j + 1, 7):
                    m = c2[j] <= c2[k]
                    rk[k] = rk[k] + jnp.where(m, one, zero)
                    rk[j] = rk[j] + jnp.where(m, zero, one)

            sa, ca = _sincos(col(rows, 3))
            sb, cb = _sincos(col(rows, 4))
            sc, cc = _sincos(col(rows, 5))
            r00 = cb * ca
            r01 = -cb * sa
            r02 = sb
            scsb = sc * sb
            ccsb = cc * sb
            r10 = cc * sa + scsb * ca
            r11 = cc * ca - scsb * sa
            r12 = -sc * cb
            r20 = sc * sa - ccsb * ca
            r21 = sc * ca + ccsb * sa
            r22 = cc * cb
            pos_x = col(rows, 0)
            pos_y = col(rows, 1)
            pos_z = col(rows, 2)

            def matvec(x, y, z):
                wx = pos_x - (r00 * x + r10 * y + r20 * z)
                wy = pos_y - (r01 * x + r11 * y + r21 * z)
                wz = pos_z - (r02 * x + r12 * y + r22 * z)
                return wx, wy, wz

            put(rows, 0, pos_x)
            put(rows, 1, pos_y)
            put(rows, 2, pos_z)
            for k in range(7):
                wx, wy, wz = matvec(px[k], py[k], pz[k])
                cbase = 3 + 3 * rk[k]
                putc(rows, cbase, wx)
                putc(rows, cbase + 1, wy)
                putc(rows, cbase + 2, wz)
            wx, wy, wz = matvec(col(rows, 62), col(rows, 63), col(rows, 64))
            put(rows, 24, wx)
            put(rows, 25, wy)
            put(rows, 26, wz)

        out_h[blk] = start_out(blk)
        if blk + 2 < NBLK:
            in_h[blk + 2] = start_in(blk + 2)

    out_h[NBLK - 2].wait()
    out_h[NBLK - 1].wait()


@jax.jit
def kernel(obs):
    mesh = plsc.VectorSubcoreMesh(core_axis_name="c", subcore_axis_name="s")
    f = functools.partial(
        pl.kernel,
        mesh=mesh,
        out_type=jax.ShapeDtypeStruct((B, D_OUT), jnp.float32),
        scratch_types=[
            pltpu.VMEM((SUB, D_IN), jnp.float32),
            pltpu.VMEM((SUB, D_IN), jnp.float32),
            pltpu.VMEM((SUB, D_OUT), jnp.float32),
            pltpu.VMEM((SUB, D_OUT), jnp.float32),
            pltpu.SemaphoreType.DMA,
            pltpu.SemaphoreType.DMA,
            pltpu.SemaphoreType.DMA,
            pltpu.SemaphoreType.DMA,
        ],
        compiler_params=pltpu.CompilerParams(needs_layout_passes=False, use_tc_tiling_on_sc=True),
    )(_mixer_body)
    return f(obs)
